# Initial kernel scaffold; baseline (speedup 1.0000x reference)
import jax
import jax.numpy as jnp
from jax.experimental import pallas as pl


def kernel(x, edge_index, edge_w, W1, W2, Wfc1, Wfc2):
    raise NotImplementedError("write your pallas kernel here")


# speedup vs baseline: 9.8755x; 9.8755x over previous
import functools

import jax
import jax.numpy as jnp
from jax import lax
from jax.experimental import pallas as pl
from jax.experimental.pallas import tpu as pltpu
from jax.experimental.pallas import tpu_sc as plsc

N = 10000
E = 320000
D = 128

NCORES = 2
NSUB = 16
NW = NCORES * NSUB
EPW = E // NW
EC = 80
NCHUNK = EPW // EC
NPAD = 10240
RPT = NPAD // NSUB

_sc_mesh = plsc.VectorSubcoreMesh(core_axis_name="c", subcore_axis_name="s")


@functools.partial(
    pl.kernel,
    mesh=_sc_mesh,
    out_type=jax.ShapeDtypeStruct((NCORES, NPAD), jnp.float32),
    scratch_types=[
        pltpu.VMEM((EC,), jnp.int32),
        pltpu.VMEM((EC,), jnp.float32),
        pltpu.VMEM_SHARED((NPAD,), jnp.float32),
    ],
)
def _sc_deg(dst_hbm, w_hbm, out_hbm, didx, wv, acc):
    c = lax.axis_index("c")
    s = lax.axis_index("s")
    for j in range(EC // 16):
        wv[pl.ds(j * 16, 16)] = jnp.zeros((16,), jnp.float32)
    for k in range(RPT // EC):
        pltpu.sync_copy(wv, acc.at[pl.ds(s * RPT + k * EC, EC)])
    plsc.subcore_barrier()
    base = (c * NSUB + s) * EPW

    def body(k, carry):
        e0 = base + k * EC
        pltpu.sync_copy(dst_hbm.at[pl.ds(e0, EC)], didx)
        pltpu.sync_copy(w_hbm.at[pl.ds(e0, EC)], wv)
        pltpu.sync_copy(wv, acc.at[didx], add=True)
        return carry

    lax.fori_loop(0, NCHUNK, body, 0)
    plsc.subcore_barrier()
    pltpu.sync_copy(acc.at[pl.ds(s * RPT, RPT)], out_hbm.at[c, pl.ds(s * RPT, RPT)])


@functools.partial(
    pl.kernel,
    mesh=_sc_mesh,
    out_type=jax.ShapeDtypeStruct((NCORES, NPAD, D), jnp.float32),
    scratch_types=[
        pltpu.VMEM((EC,), jnp.int32),
        pltpu.VMEM((EC,), jnp.int32),
        pltpu.VMEM((EC,), jnp.float32),
        pltpu.VMEM((EC, D), jnp.float32),
        pltpu.VMEM_SHARED((NPAD, D), jnp.float32),
        pltpu.SemaphoreType.DMA,
    ],
)
def _sc_agg(src_hbm, dst_hbm, w_hbm, hs_hbm, out_hbm, sidx, didx, wv, rows, acc, sem):
    c = lax.axis_index("c")
    s = lax.axis_index("s")

    def zrow(i, carry):
        for j in range(D // 16):
            rows[i, pl.ds(j * 16, 16)] = jnp.zeros((16,), jnp.float32)
        return carry

    lax.fori_loop(0, EC, zrow, 0)
    for k in range(RPT // EC):
        pltpu.sync_copy(rows, acc.at[pl.ds(s * RPT + k * EC, EC)])
    plsc.subcore_barrier()

    base = (c * NSUB + s) * EPW

    def body(k, carry):
        e0 = base + k * EC
        pltpu.sync_copy(src_hbm.at[pl.ds(e0, EC)], sidx)
        pltpu.sync_copy(dst_hbm.at[pl.ds(e0, EC)], didx)
        pltpu.sync_copy(w_hbm.at[pl.ds(e0, EC)], wv)
        pltpu.async_copy(hs_hbm.at[sidx], rows, sem).wait()

        def scale(g, carry2):
            wvec = wv[pl.ds(g * 16, 16)]
            for lane in range(16):
                i = g * 16 + lane
                wi = wvec[lane]
                for j in range(D // 16):
                    rows[i, pl.ds(j * 16, 16)] = rows[i, pl.ds(j * 16, 16)] * wi
            return carry2

        lax.fori_loop(0, EC // 16, scale, 0)
        pltpu.sync_copy(rows, acc.at[didx], add=True)
        return carry

    lax.fori_loop(0, NCHUNK, body, 0)
    plsc.subcore_barrier()
    pltpu.sync_copy(acc.at[pl.ds(s * RPT, RPT)], out_hbm.at[c, pl.ds(s * RPT, RPT)])


BN = 2000
_GRID = N // BN


def _tc1_body(x_ref, w1_ref, d0_ref, d1_ref, hs_ref, dis_ref):
    deg = 1.0 + d0_ref[0] + d1_ref[0]
    dis = lax.rsqrt(deg)
    lin = jnp.dot(x_ref[...], w1_ref[...], preferred_element_type=jnp.float32)
    hs_ref[...] = lin * dis
    dis_ref[...] = dis


def _tc1_call(x, W1, degp):
    return pl.pallas_call(
        _tc1_body,
        grid=(_GRID,),
        in_specs=[
            pl.BlockSpec((BN, D), lambda i: (i, 0)),
            pl.BlockSpec((D, D), lambda i: (0, 0)),
            pl.BlockSpec((1, BN, 1), lambda i: (0, i, 0)),
            pl.BlockSpec((1, BN, 1), lambda i: (1, i, 0)),
        ],
        out_specs=[
            pl.BlockSpec((BN, D), lambda i: (i, 0)),
            pl.BlockSpec((BN, 1), lambda i: (i, 0)),
        ],
        out_shape=[
            jax.ShapeDtypeStruct((N, D), jnp.float32),
            jax.ShapeDtypeStruct((N, 1), jnp.float32),
        ],
    )(x, W1, degp, degp)


def _leaky(v):
    return jnp.where(v >= 0, v, 0.01 * v)


def _tc2_body(a0_ref, a1_ref, hs_ref, dis_ref, w2_ref, out_ref):
    dis = dis_ref[...]
    h = _leaky(dis * (a0_ref[0] + a1_ref[0] + hs_ref[...]))
    out_ref[...] = jnp.dot(h, w2_ref[...], preferred_element_type=jnp.float32) * dis


def _tc2_call(aggp, hs1, dis, W2):
    return pl.pallas_call(
        _tc2_body,
        grid=(_GRID,),
        in_specs=[
            pl.BlockSpec((1, BN, D), lambda i: (0, i, 0)),
            pl.BlockSpec((1, BN, D), lambda i: (1, i, 0)),
            pl.BlockSpec((BN, D), lambda i: (i, 0)),
            pl.BlockSpec((BN, 1), lambda i: (i, 0)),
            pl.BlockSpec((D, D), lambda i: (0, 0)),
        ],
        out_specs=pl.BlockSpec((BN, D), lambda i: (i, 0)),
        out_shape=jax.ShapeDtypeStruct((N, D), jnp.float32),
    )(aggp, aggp, hs1, dis, W2)


def _tc3_body(a0_ref, a1_ref, hs_ref, dis_ref, wf1_ref, wf2_ref, out_ref):
    h = _leaky(dis_ref[...] * (a0_ref[0] + a1_ref[0] + hs_ref[...]))
    y = _leaky(jnp.dot(h, wf1_ref[...], preferred_element_type=jnp.float32))
    y = _leaky(jnp.dot(y, wf2_ref[...], preferred_element_type=jnp.float32))
    m = jnp.max(y, axis=1, keepdims=True)
    e = jnp.exp(y - m)
    out_ref[...] = e / jnp.sum(e, axis=1, keepdims=True)


def _tc3_call(aggp, hs2, dis, Wfc1, Wfc2):
    return pl.pallas_call(
        _tc3_body,
        grid=(_GRID,),
        in_specs=[
            pl.BlockSpec((1, BN, D), lambda i: (0, i, 0)),
            pl.BlockSpec((1, BN, D), lambda i: (1, i, 0)),
            pl.BlockSpec((BN, D), lambda i: (i, 0)),
            pl.BlockSpec((BN, 1), lambda i: (i, 0)),
            pl.BlockSpec((D, 64), lambda i: (0, 0)),
            pl.BlockSpec((64, 8), lambda i: (0, 0)),
        ],
        out_specs=pl.BlockSpec((BN, 8), lambda i: (i, 0)),
        out_shape=jax.ShapeDtypeStruct((N, 8), jnp.float32),
    )(aggp, aggp, hs2, dis, Wfc1, Wfc2)


def kernel(x, edge_index, edge_w, W1, W2, Wfc1, Wfc2):
    src = edge_index[0]
    dst = edge_index[1]
    degp = _sc_deg(dst, edge_w)[..., None]
    hs1, dis = _tc1_call(x, W1, degp)
    aggp1 = _sc_agg(src, dst, edge_w, hs1)
    hs2 = _tc2_call(aggp1, hs1, dis, W2)
    aggp2 = _sc_agg(src, dst, edge_w, hs2)
    return _tc3_call(aggp2, hs2, dis, Wfc1, Wfc2)


# speedup vs baseline: 13.2718x; 1.3439x over previous
import functools

import jax
import jax.numpy as jnp
from jax import lax
from jax.experimental import pallas as pl
from jax.experimental.pallas import tpu as pltpu
from jax.experimental.pallas import tpu_sc as plsc

N = 10000
E = 320000
D = 128

NCORES = 2
NSUB = 16
NW = NCORES * NSUB
EPW = E // NW
EC = 80
NCHUNK = EPW // EC
NPAD = 10240
RPT = NPAD // NSUB

_sc_mesh = plsc.VectorSubcoreMesh(core_axis_name="c", subcore_axis_name="s")


_DEGQ = 8


@functools.partial(
    pl.kernel,
    mesh=_sc_mesh,
    out_type=jax.ShapeDtypeStruct((NCORES, NPAD), jnp.float32),
    scratch_types=[
        pltpu.VMEM((EPW,), jnp.float32),
        pltpu.VMEM((EC,), jnp.int32),
        pltpu.VMEM((EC,), jnp.float32),
        pltpu.VMEM_SHARED((NPAD,), jnp.float32),
        pltpu.SemaphoreType.DMA,
    ],
)
def _sc_deg(dst_hbm, w_hbm, out_hbm, wb, didx, zbuf, acc, sem):
    c = lax.axis_index("c")
    s = lax.axis_index("s")
    for j in range(EC // 16):
        zbuf[pl.ds(j * 16, 16)] = jnp.zeros((16,), jnp.float32)
    for k in range(RPT // EC):
        pltpu.sync_copy(zbuf, acc.at[pl.ds(s * RPT + k * EC, EC)])
    plsc.subcore_barrier()

    base = (c * NSUB + s) * EPW
    pltpu.sync_copy(w_hbm.at[pl.ds(base, EPW)], wb)

    def body(k, cy):
        pltpu.sync_copy(dst_hbm.at[pl.ds(base + k * EC, EC)], didx)
        pltpu.sync_copy(wb.at[pl.ds(k * EC, EC)], acc.at[didx], add=True)
        return cy

    lax.fori_loop(0, NCHUNK, body, 0)
    plsc.subcore_barrier()
    pltpu.sync_copy(acc.at[pl.ds(s * RPT, RPT)], out_hbm.at[c, pl.ds(s * RPT, RPT)])


@functools.partial(
    pl.kernel,
    mesh=_sc_mesh,
    out_type=jax.ShapeDtypeStruct((NCORES, NPAD, D), jnp.float32),
    scratch_types=[
        pltpu.VMEM((EPW,), jnp.int32),
        pltpu.VMEM((EPW,), jnp.float32),
        pltpu.VMEM((EC,), jnp.int32),
        pltpu.VMEM((EC, D), jnp.float32),
        pltpu.VMEM_SHARED((NPAD, D), jnp.float32),
        pltpu.SemaphoreType.DMA,
    ],
)
def _sc_agg(src_hbm, dst_hbm, w_hbm, hs_hbm, out_hbm,
            srcb, wb, didx, rows, acc, sem):
    c = lax.axis_index("c")
    s = lax.axis_index("s")

    def zrow(i, cy):
        for j in range(D // 16):
            rows[i, pl.ds(j * 16, 16)] = jnp.zeros((16,), jnp.float32)
        return cy

    lax.fori_loop(0, EC, zrow, 0)
    for k in range(RPT // EC):
        pltpu.sync_copy(rows, acc.at[pl.ds(s * RPT + k * EC, EC)])
    plsc.subcore_barrier()

    base = (c * NSUB + s) * EPW
    pltpu.sync_copy(src_hbm.at[pl.ds(base, EPW)], srcb)
    pltpu.sync_copy(w_hbm.at[pl.ds(base, EPW)], wb)

    def body(k, cy):
        pltpu.sync_copy(dst_hbm.at[pl.ds(base + k * EC, EC)], didx)
        pltpu.async_copy(hs_hbm.at[srcb.at[pl.ds(k * EC, EC)]], rows, sem).wait()

        def group(g, cy2):
            wvec = wb[pl.ds(k * EC + g * 16, 16)]
            for lane in range(16):
                i = g * 16 + lane
                wi = wvec[lane]
                for j in range(D // 16):
                    rows[i, pl.ds(j * 16, 16)] = rows[i, pl.ds(j * 16, 16)] * wi
            return cy2

        lax.fori_loop(0, EC // 16, group, 0)
        pltpu.sync_copy(rows, acc.at[didx], add=True)
        return cy

    lax.fori_loop(0, NCHUNK, body, 0)
    plsc.subcore_barrier()
    pltpu.sync_copy(acc.at[pl.ds(s * RPT, RPT)], out_hbm.at[c, pl.ds(s * RPT, RPT)])


BN = 2000
_GRID = N // BN


def _tc1_body(x_ref, w1_ref, d0_ref, d1_ref, hs_ref, dis_ref):
    deg = 1.0 + d0_ref[0] + d1_ref[0]
    dis = lax.rsqrt(deg)
    lin = jnp.dot(x_ref[...], w1_ref[...], preferred_element_type=jnp.float32)
    hs_ref[...] = lin * dis
    dis_ref[...] = dis


def _tc1_call(x, W1, degp):
    return pl.pallas_call(
        _tc1_body,
        grid=(_GRID,),
        in_specs=[
            pl.BlockSpec((BN, D), lambda i: (i, 0)),
            pl.BlockSpec((D, D), lambda i: (0, 0)),
            pl.BlockSpec((1, BN, 1), lambda i: (0, i, 0)),
            pl.BlockSpec((1, BN, 1), lambda i: (1, i, 0)),
        ],
        out_specs=[
            pl.BlockSpec((BN, D), lambda i: (i, 0)),
            pl.BlockSpec((BN, 1), lambda i: (i, 0)),
        ],
        out_shape=[
            jax.ShapeDtypeStruct((N, D), jnp.float32),
            jax.ShapeDtypeStruct((N, 1), jnp.float32),
        ],
    )(x, W1, degp, degp)


def _leaky(v):
    return jnp.where(v >= 0, v, 0.01 * v)


def _tc2_body(a0_ref, a1_ref, hs_ref, dis_ref, w2_ref, out_ref):
    dis = dis_ref[...]
    h = _leaky(dis * (a0_ref[0] + a1_ref[0] + hs_ref[...]))
    out_ref[...] = jnp.dot(h, w2_ref[...], preferred_element_type=jnp.float32) * dis


def _tc2_call(aggp, hs1, dis, W2):
    return pl.pallas_call(
        _tc2_body,
        grid=(_GRID,),
        in_specs=[
            pl.BlockSpec((1, BN, D), lambda i: (0, i, 0)),
            pl.BlockSpec((1, BN, D), lambda i: (1, i, 0)),
            pl.BlockSpec((BN, D), lambda i: (i, 0)),
            pl.BlockSpec((BN, 1), lambda i: (i, 0)),
            pl.BlockSpec((D, D), lambda i: (0, 0)),
        ],
        out_specs=pl.BlockSpec((BN, D), lambda i: (i, 0)),
        out_shape=jax.ShapeDtypeStruct((N, D), jnp.float32),
    )(aggp, aggp, hs1, dis, W2)


def _tc3_body(a0_ref, a1_ref, hs_ref, dis_ref, wf1_ref, wf2_ref, out_ref):
    h = _leaky(dis_ref[...] * (a0_ref[0] + a1_ref[0] + hs_ref[...]))
    y = _leaky(jnp.dot(h, wf1_ref[...], preferred_element_type=jnp.float32))
    y = _leaky(jnp.dot(y, wf2_ref[...], preferred_element_type=jnp.float32))
    m = jnp.max(y, axis=1, keepdims=True)
    e = jnp.exp(y - m)
    out_ref[...] = e / jnp.sum(e, axis=1, keepdims=True)


def _tc3_call(aggp, hs2, dis, Wfc1, Wfc2):
    return pl.pallas_call(
        _tc3_body,
        grid=(_GRID,),
        in_specs=[
            pl.BlockSpec((1, BN, D), lambda i: (0, i, 0)),
            pl.BlockSpec((1, BN, D), lambda i: (1, i, 0)),
            pl.BlockSpec((BN, D), lambda i: (i, 0)),
            pl.BlockSpec((BN, 1), lambda i: (i, 0)),
            pl.BlockSpec((D, 64), lambda i: (0, 0)),
            pl.BlockSpec((64, 8), lambda i: (0, 0)),
        ],
        out_specs=pl.BlockSpec((BN, 8), lambda i: (i, 0)),
        out_shape=jax.ShapeDtypeStruct((N, 8), jnp.float32),
    )(aggp, aggp, hs2, dis, Wfc1, Wfc2)


def kernel(x, edge_index, edge_w, W1, W2, Wfc1, Wfc2):
    src = edge_index[0]
    dst = edge_index[1]
    degp = _sc_deg(dst, edge_w)[..., None]
    hs1, dis = _tc1_call(x, W1, degp)
    aggp1 = _sc_agg(src, dst, edge_w, hs1)
    hs2 = _tc2_call(aggp1, hs1, dis, W2)
    aggp2 = _sc_agg(src, dst, edge_w, hs2)
    return _tc3_call(aggp2, hs2, dis, Wfc1, Wfc2)


# speedup vs baseline: 22.6548x; 1.7070x over previous
import functools

import jax
import jax.numpy as jnp
from jax import lax
from jax.experimental import pallas as pl
from jax.experimental.pallas import tpu as pltpu
from jax.experimental.pallas import tpu_sc as plsc

N = 10000
E = 320000
D = 128

NCORES = 2
NSUB = 16
NW = NCORES * NSUB
EPW = E // NW
EC = 80
NCHUNK = EPW // EC
ECA = 128
EPWA = 10112
NCHA = EPWA // ECA
EA = NW * EPWA
NPAD = 10240
RPT = NPAD // NSUB

_sc_mesh = plsc.VectorSubcoreMesh(core_axis_name="c", subcore_axis_name="s")


@functools.partial(
    pl.kernel,
    mesh=_sc_mesh,
    out_type=jax.ShapeDtypeStruct((NCORES, NPAD), jnp.float32),
    scratch_types=[
        pltpu.VMEM((EPW,), jnp.float32),
        pltpu.VMEM((4, EC), jnp.int32),
        pltpu.VMEM((EC,), jnp.float32),
        pltpu.VMEM_SHARED((NPAD,), jnp.float32),
        pltpu.SemaphoreType.DMA((4,)),
    ],
)
def _sc_deg(dst_hbm, w_hbm, out_hbm, wb, didx, zbuf, acc, sems):
    c = lax.axis_index("c")
    s = lax.axis_index("s")
    for j in range(EC // 16):
        zbuf[pl.ds(j * 16, 16)] = jnp.zeros((16,), jnp.float32)
    for k in range(RPT // EC):
        pltpu.sync_copy(zbuf, acc.at[pl.ds(s * RPT + k * EC, EC)])
    plsc.subcore_barrier()

    base = (c * NSUB + s) * EPW
    pltpu.sync_copy(w_hbm.at[pl.ds(base, EPW)], wb)

    def proc(k, b):
        @pl.when(k >= 4)
        def _d():
            pltpu.make_async_copy(wb.at[pl.ds(k * EC, EC)],
                                  acc.at[didx.at[b]], sems.at[b]).wait()

        pltpu.sync_copy(dst_hbm.at[pl.ds(base + k * EC, EC)], didx.at[b])
        pltpu.async_copy(wb.at[pl.ds(k * EC, EC)], acc.at[didx.at[b]],
                         sems.at[b], add=True)
        del _d

    def body(g, cy):
        for b in range(4):
            proc(4 * g + b, b)
        return cy

    lax.fori_loop(0, NCHUNK // 4, body, 0)
    proc(NCHUNK - 1, 0)
    for b in (1, 2, 3, 0):
        pltpu.make_async_copy(wb.at[pl.ds(0, EC)], acc.at[didx.at[b]],
                              sems.at[b]).wait()
    plsc.subcore_barrier()
    pltpu.sync_copy(acc.at[pl.ds(s * RPT, RPT)], out_hbm.at[c, pl.ds(s * RPT, RPT)])


@functools.partial(
    pl.kernel,
    mesh=_sc_mesh,
    out_type=jax.ShapeDtypeStruct((NCORES, NPAD, D), jnp.float32),
    scratch_types=[
        pltpu.VMEM((2 * EPW,), jnp.int32),
        pltpu.VMEM((2, EC), jnp.int32),
        pltpu.VMEM((2, EC, D), jnp.float32),
        pltpu.VMEM_SHARED((NPAD, D), jnp.float32),
        pltpu.SemaphoreType.DMA((2,)),
    ],
)
def _sc_agg(sw_hbm, dst_hbm, hs_hbm, out_hbm, swb, didx, grows, acc, sems):
    c = lax.axis_index("c")
    s = lax.axis_index("s")

    def zrow(i, cy):
        for j in range(D // 16):
            grows[0, i, pl.ds(j * 16, 16)] = jnp.zeros((16,), jnp.float32)
        return cy

    lax.fori_loop(0, EC, zrow, 0)

    def zcopy(k, cy):
        pltpu.sync_copy(grows.at[0], acc.at[pl.ds(s * RPT + k * EC, EC)])
        return cy

    lax.fori_loop(0, RPT // EC, zcopy, 0)
    plsc.subcore_barrier()

    wid = c * NSUB + s
    pltpu.sync_copy(sw_hbm.at[pl.ds(wid * 2 * EPW, 2 * EPW)], swb)

    def proc(k, b):
        pltpu.make_async_copy(hs_hbm.at[swb.at[pl.ds(k * EC, EC)]],
                              grows.at[b], sems.at[b]).wait()
        pltpu.make_async_copy(dst_hbm.at[pl.ds(wid * EPW + k * EC, EC)],
                              didx.at[b], sems.at[b]).wait()

        @pl.when(k + 1 < NCHUNK)
        def _fg():
            pltpu.async_copy(hs_hbm.at[swb.at[pl.ds((k + 1) * EC, EC)]],
                             grows.at[1 - b], sems.at[1 - b])
            pltpu.async_copy(dst_hbm.at[pl.ds(wid * EPW + (k + 1) * EC, EC)],
                             didx.at[1 - b], sems.at[1 - b])

        def group(g, cy2):
            wvec = lax.bitcast_convert_type(
                swb[pl.ds(EPW + k * EC + g * 16, 16)], jnp.float32)
            for lane in range(16):
                i = g * 16 + lane
                wi = wvec[lane]
                for j in range(D // 16):
                    grows[b, i, pl.ds(j * 16, 16)] = (
                        grows[b, i, pl.ds(j * 16, 16)] * wi)
            return cy2

        lax.fori_loop(0, EC // 16, group, 0)
        pltpu.sync_copy(grows.at[b], acc.at[didx.at[b]], add=True)
        del _fg

    pltpu.async_copy(hs_hbm.at[swb.at[pl.ds(0, EC)]], grows.at[0], sems.at[0])
    pltpu.async_copy(dst_hbm.at[pl.ds(wid * EPW, EC)], didx.at[0], sems.at[0])

    def body(g, cy):
        proc(2 * g, 0)
        proc(2 * g + 1, 1)
        return cy

    lax.fori_loop(0, NCHUNK // 2, body, 0)
    proc(NCHUNK - 1, 0)
    plsc.subcore_barrier()
    pltpu.sync_copy(acc.at[pl.ds(s * RPT, RPT)], out_hbm.at[c, pl.ds(s * RPT, RPT)])


BN = 2000
_GRID = N // BN


def _tc1_body(x_ref, w1_ref, d0_ref, d1_ref, hs_ref, dis_ref):
    deg = 1.0 + d0_ref[0] + d1_ref[0]
    dis = lax.rsqrt(deg)
    lin = jnp.dot(x_ref[...], w1_ref[...], preferred_element_type=jnp.float32)
    hs_ref[...] = lin * dis
    dis_ref[...] = dis


def _tc1_call(x, W1, degp):
    return pl.pallas_call(
        _tc1_body,
        grid=(_GRID,),
        in_specs=[
            pl.BlockSpec((BN, D), lambda i: (i, 0)),
            pl.BlockSpec((D, D), lambda i: (0, 0)),
            pl.BlockSpec((1, BN, 1), lambda i: (0, i, 0)),
            pl.BlockSpec((1, BN, 1), lambda i: (1, i, 0)),
        ],
        out_specs=[
            pl.BlockSpec((BN, D), lambda i: (i, 0)),
            pl.BlockSpec((BN, 1), lambda i: (i, 0)),
        ],
        out_shape=[
            jax.ShapeDtypeStruct((N, D), jnp.float32),
            jax.ShapeDtypeStruct((N, 1), jnp.float32),
        ],
    )(x, W1, degp, degp)


def _leaky(v):
    return jnp.where(v >= 0, v, 0.01 * v)


def _tc2_body(a0_ref, a1_ref, hs_ref, dis_ref, w2_ref, out_ref):
    dis = dis_ref[...]
    h = _leaky(dis * (a0_ref[0] + a1_ref[0] + hs_ref[...]))
    out_ref[...] = jnp.dot(h, w2_ref[...], preferred_element_type=jnp.float32) * dis


def _tc2_call(aggp, hs1, dis, W2):
    return pl.pallas_call(
        _tc2_body,
        grid=(_GRID,),
        in_specs=[
            pl.BlockSpec((1, BN, D), lambda i: (0, i, 0)),
            pl.BlockSpec((1, BN, D), lambda i: (1, i, 0)),
            pl.BlockSpec((BN, D), lambda i: (i, 0)),
            pl.BlockSpec((BN, 1), lambda i: (i, 0)),
            pl.BlockSpec((D, D), lambda i: (0, 0)),
        ],
        out_specs=pl.BlockSpec((BN, D), lambda i: (i, 0)),
        out_shape=jax.ShapeDtypeStruct((N, D), jnp.float32),
    )(aggp, aggp, hs1, dis, W2)


def _tc3_body(a0_ref, a1_ref, hs_ref, dis_ref, wf1_ref, wf2_ref, out_ref):
    h = _leaky(dis_ref[...] * (a0_ref[0] + a1_ref[0] + hs_ref[...]))
    y = _leaky(jnp.dot(h, wf1_ref[...], preferred_element_type=jnp.float32))
    y = _leaky(jnp.dot(y, wf2_ref[...], preferred_element_type=jnp.float32))
    m = jnp.max(y, axis=1, keepdims=True)
    e = jnp.exp(y - m)
    out_ref[...] = e / jnp.sum(e, axis=1, keepdims=True)


def _tc3_call(aggp, hs2, dis, Wfc1, Wfc2):
    return pl.pallas_call(
        _tc3_body,
        grid=(_GRID,),
        in_specs=[
            pl.BlockSpec((1, BN, D), lambda i: (0, i, 0)),
            pl.BlockSpec((1, BN, D), lambda i: (1, i, 0)),
            pl.BlockSpec((BN, D), lambda i: (i, 0)),
            pl.BlockSpec((BN, 1), lambda i: (i, 0)),
            pl.BlockSpec((D, 64), lambda i: (0, 0)),
            pl.BlockSpec((64, 8), lambda i: (0, 0)),
        ],
        out_specs=pl.BlockSpec((BN, 8), lambda i: (i, 0)),
        out_shape=jax.ShapeDtypeStruct((N, 8), jnp.float32),
    )(aggp, aggp, hs2, dis, Wfc1, Wfc2)


def kernel(x, edge_index, edge_w, W1, W2, Wfc1, Wfc2):
    src = edge_index[0]
    dst = edge_index[1]
    wbits = jax.lax.bitcast_convert_type(edge_w, jnp.int32)
    sw = jnp.concatenate(
        [src.reshape(NW, EPW), wbits.reshape(NW, EPW)], axis=1).reshape(-1)
    degp = _sc_deg(dst, edge_w)[..., None]
    hs1, dis = _tc1_call(x, W1, degp)
    aggp1 = _sc_agg(sw, dst, hs1)
    hs2 = _tc2_call(aggp1, hs1, dis, W2)
    aggp2 = _sc_agg(sw, dst, hs2)
    return _tc3_call(aggp2, hs2, dis, Wfc1, Wfc2)
